# baseline (speedup 1.0000x reference)
import functools

import jax
import jax.numpy as jnp
from jax import lax
from jax.experimental import pallas as pl
from jax.experimental.pallas import tpu as pltpu
from jax.experimental.pallas import tpu_sc as plsc

NC, NS = 2, 16
NW = NC * NS
L = 16
C = 4096


@functools.lru_cache(maxsize=None)
def _build(M, D, B):
    assert D % NW == 0 and M % (2 * L) == 0
    H = M // 2
    cols_per_w = D // NW
    ntask = cols_per_w * 2
    nchunk = B // C
    assert B % C == 0 and C % L == 0 and nchunk >= 2
    mesh = plsc.VectorSubcoreMesh(
        core_axis_name="c", subcore_axis_name="s", num_cores=NC, num_subcores=NS
    )

    @functools.partial(
        pl.kernel,
        out_type=jax.ShapeDtypeStruct((D * M,), jnp.float32),
        mesh=mesh,
        compiler_params=pltpu.CompilerParams(needs_layout_passes=False),
        scratch_types=[
            pltpu.VMEM((H,), jnp.float32),
            pltpu.VMEM((H,), jnp.float32),
            pltpu.VMEM((B,), jnp.int32),
            pltpu.VMEM((2, C), jnp.float32),
            pltpu.SemaphoreType.DMA,
            pltpu.SemaphoreType.DMA,
            pltpu.SemaphoreType.DMA,
            pltpu.SemaphoreType.DMA,
            pltpu.SemaphoreType.DMA,
            pltpu.SemaphoreType.DMA,
            pltpu.SemaphoreType.DMA,
        ],
    )
    def scatter_cols(
        inp_f, idx_t, src_t, out_f, buf0, buf1, idx_v, src2,
        sem_l0, sem_l1, sem_o0, sem_o1, sem_i, sem_a, sem_b,
    ):
        wid = lax.axis_index("s") * NC + lax.axis_index("c")
        bufs = (buf0, buf1)
        sem_l = (sem_l0, sem_l1)
        sem_o = (sem_o0, sem_o1)
        src_sems = (sem_a, sem_b)

        def task_col(t):
            return (t // 2) * NW + wid

        def start_src(col, c):
            return pltpu.async_copy(
                src_t.at[col, pl.ds(c * C, C)], src2.at[c % 2], src_sems[c % 2]
            )

        def half_base(t):
            return pl.multiple_of(task_col(t) * M + (t % 2) * H, 8)

        def start_load(t):
            b = t % 2
            return pltpu.async_copy(
                inp_f.at[pl.ds(half_base(t), H)], bufs[b], sem_l[b]
            )

        ld = {0: start_load(0), 1: start_load(1)}
        idx_cp = pltpu.async_copy(idx_t.at[task_col(0)], idx_v, sem_i)
        src_cp = {0: start_src(task_col(0), 0)}
        st = {}

        for t in range(ntask):
            col, h, b = task_col(t), t % 2, t % 2
            ld[t].wait()
            if t % 2 == 0:
                idx_cp.wait()
            nxt0 = None
            for c in range(nchunk):
                if c + 1 < nchunk:
                    src_cp[c + 1] = start_src(col, c + 1)
                elif t + 1 < ntask:
                    nxt0 = start_src(task_col(t + 1), 0)
                src_cp[c].wait()
                if c == nchunk - 2 and (t - 1) in st:
                    st.pop(t - 1).wait()
                    if t + 1 < ntask:
                        ld[t + 1] = start_load(t + 1)

                def sweep(i, carry, c=c, h=h, b=b):
                    idx16 = idx_v[pl.ds(c * C + i * L, L)]
                    val16 = src2[c % 2, pl.ds(i * L, L)]
                    rel = idx16 - h * H
                    m = (rel >= 0) & (rel < H)
                    safe = jnp.where(m, rel, 0)
                    plsc.addupdate_scatter(bufs[b], [safe], val16, mask=m)
                    return carry

                lax.fori_loop(0, C // L, sweep, 0, unroll=8)
            if t == 0:
                pass
            st[t] = pltpu.async_copy(
                bufs[b], out_f.at[pl.ds(half_base(t), H)], sem_o[b]
            )
            if nxt0 is not None:
                src_cp = {0: nxt0}
            if t % 2 == 1 and t + 1 < ntask:
                idx_cp = pltpu.async_copy(idx_t.at[task_col(t + 1)], idx_v, sem_i)

        for t in list(st):
            st.pop(t).wait()

    return scatter_cols


def kernel(input, index, src):
    M, D = input.shape
    B = src.shape[0]
    inp_t = input.T
    idx_t = index.astype(jnp.int32).T
    src_t = src.T
    out_f = _build(M, D, B)(inp_t.reshape(-1), idx_t, src_t)
    return out_f.reshape(D, M).T


# speedup vs baseline: 2.5463x; 2.5463x over previous
import functools

import jax
import jax.numpy as jnp
from jax import lax
from jax.experimental import pallas as pl
from jax.experimental.pallas import tpu as pltpu
from jax.experimental.pallas import tpu_sc as plsc

NC, NS = 2, 16
NW = NC * NS
L = 16
C = 4096


@functools.lru_cache(maxsize=None)
def _build(M, D, B):
    assert D % NW == 0 and M % L == 0
    cols_per_w = D // NW
    nchunk = B // C
    assert B % C == 0 and C % L == 0 and nchunk >= 2
    mesh = plsc.VectorSubcoreMesh(
        core_axis_name="c", subcore_axis_name="s", num_cores=NC, num_subcores=NS
    )

    @functools.partial(
        pl.kernel,
        out_type=jax.ShapeDtypeStruct((D, M), jnp.float32),
        mesh=mesh,
        compiler_params=pltpu.CompilerParams(needs_layout_passes=False),
        scratch_types=[
            pltpu.VMEM((M,), jnp.float32),
            pltpu.VMEM((B,), jnp.int32),
            pltpu.VMEM((2, C), jnp.float32),
            pltpu.SemaphoreType.DMA,
            pltpu.SemaphoreType.DMA,
            pltpu.SemaphoreType.DMA,
            pltpu.SemaphoreType.DMA,
            pltpu.SemaphoreType.DMA,
        ],
    )
    def scatter_cols(
        inp_t, idx_t, src_t, out_t, acc_v, idx_v, src2, sem_c, sem_o, sem_i,
        sem_a, sem_b,
    ):
        wid = lax.axis_index("s") * NC + lax.axis_index("c")
        src_sems = (sem_a, sem_b)

        def start_src(col, c):
            return pltpu.async_copy(
                src_t.at[col, pl.ds(c * C, C)], src2.at[c % 2], src_sems[c % 2]
            )

        col0 = 0 * NW + wid
        col_cp = pltpu.async_copy(inp_t.at[col0], acc_v, sem_c)
        idx_cp = pltpu.async_copy(idx_t.at[col0], idx_v, sem_i)
        src_cp = {0: start_src(col0, 0)}

        for k in range(cols_per_w):
            col = k * NW + wid
            col_cp.wait()
            idx_cp.wait()
            for c in range(nchunk):
                if c + 1 < nchunk:
                    src_cp[c + 1] = start_src(col, c + 1)
                elif k + 1 < cols_per_w:
                    nxt_src = start_src(col + NW, 0)
                src_cp[c].wait()

                def scat(i, carry, c=c):
                    idx16 = idx_v[pl.ds(c * C + i * L, L)]
                    val16 = src2[c % 2, pl.ds(i * L, L)]
                    plsc.addupdate_scatter(acc_v, [idx16], val16)
                    return carry

                lax.fori_loop(0, C // L, scat, 0, unroll=8)
            st_cp = pltpu.async_copy(acc_v, out_t.at[col], sem_o)
            if k + 1 < cols_per_w:
                idx_cp = pltpu.async_copy(idx_t.at[col + NW], idx_v, sem_i)
                src_cp = {0: nxt_src}
                st_cp.wait()
                col_cp = pltpu.async_copy(inp_t.at[col + NW], acc_v, sem_c)
            else:
                st_cp.wait()

    return scatter_cols


def kernel(input, index, src):
    M, D = input.shape
    B = src.shape[0]
    inp_t = input.T
    idx_t = index.astype(jnp.int32).T
    src_t = src.T
    out_t = _build(M, D, B)(inp_t, idx_t, src_t)
    return out_t.T
